# baseline (speedup 1.0000x reference)
import functools

import jax
import jax.numpy as jnp
from jax import lax
from jax.experimental import pallas as pl
from jax.experimental.pallas import tpu as pltpu
from jax.experimental.pallas import tpu_sc as plsc

N_EMB = 8192
D_EMB = 32
N_TOK = 8192
C_COST = 0.25

T_BLK = 256
C_BLK = 2048
NT = N_TOK // T_BLK
NC = N_EMB // C_BLK

_BIG_I32 = 2 ** 30


def _k1_body(x_ref, w_ref, sx_ref, sw_ref, idx_ref, rmin_ref):
    j = pl.program_id(1)

    x = x_ref[...]
    w = w_ref[...]
    sx = sx_ref[...]
    sw = sw_ref[0:1, :]
    p = lax.dot_general(x, w, (((1,), (1,)), ((), ())),
                        preferred_element_type=jnp.float32)
    d2 = (sx - p * 2.0) + sw
    s = jnp.sqrt(jnp.maximum(d2, 0.0))
    bm = jnp.min(s, axis=1, keepdims=True)
    cols = lax.broadcasted_iota(jnp.int32, (T_BLK, C_BLK), 1) + j * C_BLK
    bi = jnp.min(jnp.where(s == bm, cols, _BIG_I32), axis=1, keepdims=True)

    def _round(v):
        return v.astype(jnp.bfloat16).astype(jnp.float32)

    @pl.when(j == 0)
    def _init():
        rmin_ref[...] = _round(bm)
        idx_ref[...] = bi

    @pl.when(j > 0)
    def _update():
        better = bm < rmin_ref[...]
        idx_ref[...] = jnp.where(better, bi, idx_ref[...])
        rmin_ref[...] = _round(jnp.where(better, bm, rmin_ref[...]))


def _argmin_tc(flat_xb, weight, sx, swb):
    idx, _ = pl.pallas_call(
        _k1_body,
        grid=(NT, NC),
        in_specs=[
            pl.BlockSpec((T_BLK, D_EMB), lambda i, j: (i, 0)),
            pl.BlockSpec((C_BLK, D_EMB), lambda i, j: (j, 0)),
            pl.BlockSpec((T_BLK, 1), lambda i, j: (i, 0)),
            pl.BlockSpec((8, C_BLK), lambda i, j: (0, j)),
        ],
        out_specs=[
            pl.BlockSpec((T_BLK, 1), lambda i, j: (i, 0)),
            pl.BlockSpec((T_BLK, 1), lambda i, j: (i, 0)),
        ],
        out_shape=[
            jax.ShapeDtypeStruct((N_TOK, 1), jnp.int32),
            jax.ShapeDtypeStruct((N_TOK, 1), jnp.float32),
        ],
    )(flat_xb, weight, sx, swb)
    return idx


_NW = 32
_TPW = N_TOK // _NW
_CH = 128
_NCH = _TPW // _CH


def _sc_body(idx_hbm, w_hbm, q_hbm, counts_hbm,
             idx_v, rows_v, ones_v, zero_v, shc, sem):
    cid = lax.axis_index("c")
    sid = lax.axis_index("s")
    wid = sid * 2 + cid
    base = wid * _NCH

    pltpu.sync_copy(idx_hbm.at[pl.ds(base, _NCH)], idx_v)
    copies = []
    for ch in range(_NCH):
        copies.append(pltpu.async_copy(
            w_hbm.at[idx_v.at[ch]], rows_v.at[pl.ds(ch * _CH, _CH)], sem))

    @pl.when(sid == 0)
    def _zero():
        def zbody(i, _):
            zero_v[pl.ds(i * 16, 16)] = jnp.zeros((16,), jnp.int32)
            return 0
        lax.fori_loop(0, N_EMB // 16, zbody, 0)
        pltpu.sync_copy(zero_v, shc)

    for k in range(8):
        ones_v[pl.ds(k * 16, 16)] = jnp.full((16,), 1, jnp.int32)

    plsc.subcore_barrier()
    for ch in range(_NCH):
        pltpu.sync_copy(ones_v, shc.at[idx_v.at[ch]], add=True)
    plsc.subcore_barrier()

    @pl.when(sid == 0)
    def _export():
        pltpu.sync_copy(shc, counts_hbm.at[cid])

    for c in copies:
        c.wait()
    pltpu.sync_copy(rows_v, q_hbm.at[pl.ds(wid * _TPW, _TPW)])


def _gather_hist_sc(idx2d, weight):
    mesh = plsc.VectorSubcoreMesh(core_axis_name="c", subcore_axis_name="s")
    fn = functools.partial(
        pl.kernel,
        mesh=mesh,
        out_type=[
            jax.ShapeDtypeStruct((N_TOK, D_EMB), jnp.float32),
            jax.ShapeDtypeStruct((2, N_EMB), jnp.int32),
        ],
        scratch_types=[
            pltpu.VMEM((_NCH, _CH), jnp.int32),
            pltpu.VMEM((_TPW, D_EMB), jnp.float32),
            pltpu.VMEM((_CH,), jnp.int32),
            pltpu.VMEM((N_EMB,), jnp.int32),
            pltpu.VMEM_SHARED((N_EMB,), jnp.int32),
            pltpu.SemaphoreType.DMA,
        ],
        compiler_params=pltpu.CompilerParams(use_tc_tiling_on_sc=False),
    )(_sc_body)
    return fn(idx2d, weight)


def _k3_body(x_ref, q_ref, cnt_ref, out_ref, loss_ref, perp_ref):
    x = x_ref[...]
    q = q_ref[...]
    diff = q - x
    out_ref[...] = x + diff
    l = jnp.mean(diff * diff)
    loss_ref[...] = jnp.reshape(l + C_COST * l, (1, 1))
    ctot = cnt_ref[0:1, :] + cnt_ref[1:2, :]
    p = ctot.astype(jnp.float32) * jnp.float32(1.0 / N_TOK)
    ent = jnp.sum(p * jnp.log(p + 1e-10))
    perp_ref[...] = jnp.reshape(jnp.exp(-ent), (1, 1))


def _finish_tc(flat_x, q, counts):
    return pl.pallas_call(
        _k3_body,
        out_shape=[
            jax.ShapeDtypeStruct((N_TOK, D_EMB), jnp.float32),
            jax.ShapeDtypeStruct((1, 1), jnp.float32),
            jax.ShapeDtypeStruct((1, 1), jnp.float32),
        ],
    )(flat_x, q, counts)


def kernel(inputs, weight):
    flat_x = jnp.transpose(inputs, (0, 2, 1)).reshape(N_TOK, D_EMB)
    flat_xb = flat_x.astype(jnp.bfloat16)
    sx = jnp.sum(flat_x * flat_x, axis=1).reshape(N_TOK, 1)
    swb = jnp.broadcast_to(jnp.sum(weight * weight, axis=1)[None, :], (8, N_EMB))
    idx = _argmin_tc(flat_xb, weight, sx, swb)
    idx_flat = idx.reshape(N_TOK)
    q, counts = _gather_hist_sc(idx.reshape(_NW * _NCH, _CH), weight)
    out_flat, loss, perp = _finish_tc(flat_x, q, counts)
    quantized_out = jnp.transpose(out_flat.reshape(8, 1024, 32), (0, 2, 1))
    enc_idx = idx_flat.reshape(8, 1024)
    return (quantized_out, loss.reshape(()), enc_idx, perp.reshape(()))


# speedup vs baseline: 1.1793x; 1.1793x over previous
import functools

import jax
import jax.numpy as jnp
from jax import lax
from jax.experimental import pallas as pl
from jax.experimental.pallas import tpu as pltpu
from jax.experimental.pallas import tpu_sc as plsc

N_EMB = 8192
D_EMB = 32
N_TOK = 8192
C_COST = 0.25

T_BLK = 1024
C_BLK = 2048
NT = N_TOK // T_BLK
NC = N_EMB // C_BLK

_BIG_I32 = 2 ** 30


def _k1_body(x_ref, w_ref, sx_ref, sw_ref, idx_ref, rmin_ref):
    j = pl.program_id(1)

    x = x_ref[...]
    w2 = w_ref[...]
    sx = sx_ref[...]
    sw = sw_ref[0:1, :]
    p2 = lax.dot_general(x, w2, (((1,), (1,)), ((), ())),
                         preferred_element_type=jnp.float32)
    d2 = (sx - p2) + sw
    s = jnp.sqrt(jnp.maximum(d2, 0.0))
    bm = jnp.min(s, axis=1, keepdims=True)
    cols = lax.broadcasted_iota(jnp.int32, (T_BLK, C_BLK), 1) + j * C_BLK
    bi = jnp.min(jnp.where(s == bm, cols, _BIG_I32), axis=1, keepdims=True)

    def _round(v):
        return v.astype(jnp.bfloat16).astype(jnp.float32)

    @pl.when(j == 0)
    def _init():
        rmin_ref[...] = _round(bm)
        idx_ref[...] = bi

    @pl.when(j > 0)
    def _update():
        better = bm < rmin_ref[...]
        idx_ref[...] = jnp.where(better, bi, idx_ref[...])
        rmin_ref[...] = _round(jnp.where(better, bm, rmin_ref[...]))


def _argmin_tc(flat_xb, weight, sx, swb):
    idx, _ = pl.pallas_call(
        _k1_body,
        grid=(NT, NC),
        in_specs=[
            pl.BlockSpec((T_BLK, D_EMB), lambda i, j: (i, 0)),
            pl.BlockSpec((C_BLK, D_EMB), lambda i, j: (j, 0)),
            pl.BlockSpec((T_BLK, 1), lambda i, j: (i, 0)),
            pl.BlockSpec((8, C_BLK), lambda i, j: (0, j)),
        ],
        out_specs=[
            pl.BlockSpec((T_BLK, 1), lambda i, j: (i, 0)),
            pl.BlockSpec((T_BLK, 1), lambda i, j: (i, 0)),
        ],
        out_shape=[
            jax.ShapeDtypeStruct((N_TOK, 1), jnp.int32),
            jax.ShapeDtypeStruct((N_TOK, 1), jnp.float32),
        ],
    )(flat_xb, weight, sx, swb)
    return idx


_NW = 32
_TPW = N_TOK // _NW
_CH = 128
_NCH = _TPW // _CH


def _sc_body(idx_hbm, w_hbm, q_hbm, counts_hbm,
             idx_v, rows_v, ones_v, zero_v, shc, sem):
    cid = lax.axis_index("c")
    sid = lax.axis_index("s")
    wid = sid * 2 + cid
    base = wid * _NCH

    pltpu.sync_copy(idx_hbm.at[pl.ds(base, _NCH)], idx_v)
    copies = []
    for ch in range(_NCH):
        copies.append(pltpu.async_copy(
            w_hbm.at[idx_v.at[ch]], rows_v.at[pl.ds(ch * _CH, _CH)], sem))

    @pl.when(sid == 0)
    def _zero():
        def zbody(i, _):
            zero_v[pl.ds(i * 16, 16)] = jnp.zeros((16,), jnp.int32)
            return 0
        lax.fori_loop(0, N_EMB // 16, zbody, 0)
        pltpu.sync_copy(zero_v, shc)

    for k in range(8):
        ones_v[pl.ds(k * 16, 16)] = jnp.full((16,), 1, jnp.int32)

    plsc.subcore_barrier()
    for ch in range(_NCH):
        pltpu.sync_copy(ones_v, shc.at[idx_v.at[ch]], add=True)
    plsc.subcore_barrier()

    @pl.when(sid == 0)
    def _export():
        pltpu.sync_copy(shc, counts_hbm.at[cid])

    for c in copies:
        c.wait()
    pltpu.sync_copy(rows_v, q_hbm.at[pl.ds(wid * _TPW, _TPW)])


def _gather_hist_sc(idx2d, weight):
    mesh = plsc.VectorSubcoreMesh(core_axis_name="c", subcore_axis_name="s")
    fn = functools.partial(
        pl.kernel,
        mesh=mesh,
        out_type=[
            jax.ShapeDtypeStruct((N_TOK, D_EMB), jnp.float32),
            jax.ShapeDtypeStruct((2, N_EMB), jnp.int32),
        ],
        scratch_types=[
            pltpu.VMEM((_NCH, _CH), jnp.int32),
            pltpu.VMEM((_TPW, D_EMB), jnp.float32),
            pltpu.VMEM((_CH,), jnp.int32),
            pltpu.VMEM((N_EMB,), jnp.int32),
            pltpu.VMEM_SHARED((N_EMB,), jnp.int32),
            pltpu.SemaphoreType.DMA,
        ],
        compiler_params=pltpu.CompilerParams(use_tc_tiling_on_sc=False),
    )(_sc_body)
    return fn(idx2d, weight)


def _k3_body(x_ref, q_ref, cnt_ref, out_ref, loss_ref, perp_ref):
    x = x_ref[...]
    q = q_ref[...]
    diff = q - x
    out_ref[...] = x + diff
    l = jnp.mean(diff * diff)
    loss_ref[...] = jnp.reshape(l + C_COST * l, (1, 1))
    ctot = cnt_ref[0:1, :] + cnt_ref[1:2, :]
    p = ctot.astype(jnp.float32) * jnp.float32(1.0 / N_TOK)
    ent = jnp.sum(p * jnp.log(p + 1e-10))
    perp_ref[...] = jnp.reshape(jnp.exp(-ent), (1, 1))


def _finish_tc(flat_x, q, counts):
    return pl.pallas_call(
        _k3_body,
        out_shape=[
            jax.ShapeDtypeStruct((N_TOK, D_EMB), jnp.float32),
            jax.ShapeDtypeStruct((1, 1), jnp.float32),
            jax.ShapeDtypeStruct((1, 1), jnp.float32),
        ],
    )(flat_x, q, counts)


def kernel(inputs, weight):
    flat_x = jnp.transpose(inputs, (0, 2, 1)).reshape(N_TOK, D_EMB)
    flat_xb = flat_x.astype(jnp.bfloat16)
    sx = jnp.sum(flat_x * flat_x, axis=1).reshape(N_TOK, 1)
    swb = jnp.broadcast_to(jnp.sum(weight * weight, axis=1)[None, :], (8, N_EMB))
    idx = _argmin_tc(flat_xb, weight + weight, sx, swb)
    idx_flat = idx.reshape(N_TOK)
    q, counts = _gather_hist_sc(idx.reshape(_NW * _NCH, _CH), weight)
    out_flat, loss, perp = _finish_tc(flat_x, q, counts)
    quantized_out = jnp.transpose(out_flat.reshape(8, 1024, 32), (0, 2, 1))
    enc_idx = idx_flat.reshape(8, 1024)
    return (quantized_out, loss.reshape(()), enc_idx, perp.reshape(()))


# speedup vs baseline: 1.2656x; 1.0732x over previous
import functools

import jax
import jax.numpy as jnp
from jax import lax
from jax.experimental import pallas as pl
from jax.experimental.pallas import tpu as pltpu
from jax.experimental.pallas import tpu_sc as plsc

N_EMB = 8192
D_EMB = 32
N_TOK = 8192
C_COST = 0.25

T_BLK = 1024
C_BLK = 2048
NT = N_TOK // T_BLK
NC = N_EMB // C_BLK

_BIG_I32 = 2 ** 30


def _k1_body(x_ref, w_ref, sx_ref, sw_ref, cols_ref, idx_ref, rmin_ref):
    j = pl.program_id(1)

    x = jnp.transpose(x_ref[0]).astype(jnp.bfloat16)
    w2 = w_ref[...]
    sx = sx_ref[...]
    sw = sw_ref[0:1, :]
    colf = cols_ref[0:1, :]
    p2 = lax.dot_general(x, w2, (((1,), (1,)), ((), ())),
                         preferred_element_type=jnp.float32)
    d2 = (sx - p2) + sw
    s = jnp.sqrt(jnp.maximum(d2, 0.0))
    bm = jnp.min(s, axis=1, keepdims=True)
    bif = jnp.min(jnp.where(s == bm, colf, jnp.float32(3e38)),
                  axis=1, keepdims=True)
    bi = bif.astype(jnp.int32)

    def _round(v):
        return v.astype(jnp.bfloat16).astype(jnp.float32)

    @pl.when(j == 0)
    def _init():
        rmin_ref[...] = _round(bm)
        idx_ref[...] = bi

    @pl.when(j > 0)
    def _update():
        better = bm < rmin_ref[...]
        idx_ref[...] = jnp.where(better, bi, idx_ref[...])
        rmin_ref[...] = _round(jnp.where(better, bm, rmin_ref[...]))


def _argmin_tc(inputs, weight, sx, swb, colsb):
    idx, _ = pl.pallas_call(
        _k1_body,
        grid=(NT, NC),
        in_specs=[
            pl.BlockSpec((1, 32, 1024), lambda i, j: (i, 0, 0)),
            pl.BlockSpec((C_BLK, D_EMB), lambda i, j: (j, 0)),
            pl.BlockSpec((T_BLK, 1), lambda i, j: (i, 0)),
            pl.BlockSpec((8, C_BLK), lambda i, j: (0, j)),
            pl.BlockSpec((8, C_BLK), lambda i, j: (0, j)),
        ],
        out_specs=[
            pl.BlockSpec((T_BLK, 1), lambda i, j: (i, 0)),
            pl.BlockSpec((T_BLK, 1), lambda i, j: (i, 0)),
        ],
        out_shape=[
            jax.ShapeDtypeStruct((N_TOK, 1), jnp.int32),
            jax.ShapeDtypeStruct((N_TOK, 1), jnp.float32),
        ],
    )(inputs, weight, sx, swb, colsb)
    return idx


_NW = 32
_TPW = N_TOK // _NW
_CH = 128
_NCH = _TPW // _CH


def _sc_body(idx_hbm, w_hbm, q_hbm, counts_hbm,
             idx_v, rows_v, ones_v, zero_v, shc, sem):
    cid = lax.axis_index("c")
    sid = lax.axis_index("s")
    wid = sid * 2 + cid
    base = wid * _NCH

    pltpu.sync_copy(idx_hbm.at[pl.ds(base, _NCH)], idx_v)
    copies = []
    for ch in range(_NCH):
        copies.append(pltpu.async_copy(
            w_hbm.at[idx_v.at[ch]], rows_v.at[pl.ds(ch * _CH, _CH)], sem))

    @pl.when(sid == 0)
    def _zero():
        def zbody(i, _):
            zero_v[pl.ds(i * 16, 16)] = jnp.zeros((16,), jnp.int32)
            return 0
        lax.fori_loop(0, N_EMB // 16, zbody, 0)
        pltpu.sync_copy(zero_v, shc)

    for k in range(8):
        ones_v[pl.ds(k * 16, 16)] = jnp.full((16,), 1, jnp.int32)

    plsc.subcore_barrier()
    for ch in range(_NCH):
        pltpu.sync_copy(ones_v, shc.at[idx_v.at[ch]], add=True)
    plsc.subcore_barrier()

    @pl.when(sid == 0)
    def _export():
        pltpu.sync_copy(shc, counts_hbm.at[cid])

    for c in copies:
        c.wait()
    pltpu.sync_copy(rows_v, q_hbm.at[pl.ds(wid * _TPW, _TPW)])


def _gather_hist_sc(idx2d, weight):
    mesh = plsc.VectorSubcoreMesh(core_axis_name="c", subcore_axis_name="s")
    fn = functools.partial(
        pl.kernel,
        mesh=mesh,
        out_type=[
            jax.ShapeDtypeStruct((N_TOK, D_EMB), jnp.float32),
            jax.ShapeDtypeStruct((2, N_EMB), jnp.int32),
        ],
        scratch_types=[
            pltpu.VMEM((_NCH, _CH), jnp.int32),
            pltpu.VMEM((_TPW, D_EMB), jnp.float32),
            pltpu.VMEM((_CH,), jnp.int32),
            pltpu.VMEM((N_EMB,), jnp.int32),
            pltpu.VMEM_SHARED((N_EMB,), jnp.int32),
            pltpu.SemaphoreType.DMA,
        ],
        compiler_params=pltpu.CompilerParams(use_tc_tiling_on_sc=False),
    )(_sc_body)
    return fn(idx2d, weight)


def _k3_body(x_ref, q_ref, cnt_ref, out_ref, loss_ref, perp_ref, acc_ref):
    b = pl.program_id(0)
    x = x_ref[0]
    qt = jnp.transpose(q_ref[...])
    diff = qt - x
    out_ref[...] = (x + diff)[None]
    part = jnp.sum(diff * diff)

    @pl.when(b == 0)
    def _init():
        acc_ref[0, 0] = part

    @pl.when(b > 0)
    def _acc():
        acc_ref[0, 0] = acc_ref[0, 0] + part

    @pl.when(b == 7)
    def _final():
        l = acc_ref[0, 0] * jnp.float32(1.0 / (N_TOK * D_EMB))
        loss_ref[...] = jnp.reshape(l + C_COST * l, (1, 1))
        ctot = cnt_ref[0:1, :] + cnt_ref[1:2, :]
        p = ctot.astype(jnp.float32) * jnp.float32(1.0 / N_TOK)
        ent = jnp.sum(p * jnp.log(p + 1e-10))
        perp_ref[...] = jnp.reshape(jnp.exp(-ent), (1, 1))


def _finish_tc(inputs, q, counts):
    return pl.pallas_call(
        _k3_body,
        grid=(8,),
        in_specs=[
            pl.BlockSpec((1, 32, 1024), lambda b: (b, 0, 0)),
            pl.BlockSpec((1024, D_EMB), lambda b: (b, 0)),
            pl.BlockSpec((2, N_EMB), lambda b: (0, 0)),
        ],
        out_specs=[
            pl.BlockSpec((1, 32, 1024), lambda b: (b, 0, 0)),
            pl.BlockSpec((1, 1), lambda b: (0, 0)),
            pl.BlockSpec((1, 1), lambda b: (0, 0)),
        ],
        out_shape=[
            jax.ShapeDtypeStruct((8, 32, 1024), jnp.float32),
            jax.ShapeDtypeStruct((1, 1), jnp.float32),
            jax.ShapeDtypeStruct((1, 1), jnp.float32),
        ],
        scratch_shapes=[pltpu.SMEM((1, 1), jnp.float32)],
    )(inputs, q, counts)


def kernel(inputs, weight):
    flat_x = jnp.transpose(inputs, (0, 2, 1)).reshape(N_TOK, D_EMB)
    sx = jnp.sum(flat_x * flat_x, axis=1).reshape(N_TOK, 1)
    swb = jnp.broadcast_to(jnp.sum(weight * weight, axis=1)[None, :], (8, N_EMB))
    colsb = jnp.broadcast_to(
        jnp.arange(N_EMB, dtype=jnp.float32)[None, :], (8, N_EMB))
    idx = _argmin_tc(inputs, weight + weight, sx, swb, colsb)
    q, counts = _gather_hist_sc(idx.reshape(_NW * _NCH, _CH), weight)
    quantized_out, loss, perp = _finish_tc(inputs, q, counts)
    enc_idx = idx.reshape(8, 1024)
    return (quantized_out, loss.reshape(()), enc_idx, perp.reshape(()))


# speedup vs baseline: 1.2920x; 1.0208x over previous
import functools

import jax
import jax.numpy as jnp
from jax import lax
from jax.experimental import pallas as pl
from jax.experimental.pallas import tpu as pltpu
from jax.experimental.pallas import tpu_sc as plsc

N_EMB = 8192
D_EMB = 32
N_TOK = 8192
C_COST = 0.25

T_BLK = 2048
C_BLK = 2048
NT = N_TOK // T_BLK
NC = N_EMB // C_BLK

_BIG_I32 = 2 ** 30


def _k1_body(x_ref, w_ref, sx_ref, sw_ref, cols_ref, idx_ref, rmin_ref):
    j = pl.program_id(1)

    x = jnp.concatenate(
        [jnp.transpose(x_ref[k]) for k in range(T_BLK // 1024)],
        axis=0).astype(jnp.bfloat16)
    w2 = w_ref[...]
    sx = sx_ref[...]
    sw = sw_ref[0:1, :]
    colf = cols_ref[0:1, :]
    p2 = lax.dot_general(x, w2, (((1,), (1,)), ((), ())),
                         preferred_element_type=jnp.float32)
    d2 = (sx - p2) + sw
    s = jnp.sqrt(jnp.maximum(d2, 0.0))
    bm = jnp.min(s, axis=1, keepdims=True)
    bif = jnp.min(jnp.where(s == bm, colf, jnp.float32(3e38)),
                  axis=1, keepdims=True)
    bi = bif.astype(jnp.int32)

    def _round(v):
        return v.astype(jnp.bfloat16).astype(jnp.float32)

    @pl.when(j == 0)
    def _init():
        rmin_ref[...] = _round(bm)
        idx_ref[...] = bi

    @pl.when(j > 0)
    def _update():
        better = bm < rmin_ref[...]
        idx_ref[...] = jnp.where(better, bi, idx_ref[...])
        rmin_ref[...] = _round(jnp.where(better, bm, rmin_ref[...]))


def _argmin_tc(inputs, weight, sx, swb, colsb):
    idx, _ = pl.pallas_call(
        _k1_body,
        grid=(NT, NC),
        in_specs=[
            pl.BlockSpec((2, 32, 1024), lambda i, j: (i, 0, 0)),
            pl.BlockSpec((C_BLK, D_EMB), lambda i, j: (j, 0)),
            pl.BlockSpec((T_BLK, 1), lambda i, j: (i, 0)),
            pl.BlockSpec((8, C_BLK), lambda i, j: (0, j)),
            pl.BlockSpec((8, C_BLK), lambda i, j: (0, j)),
        ],
        out_specs=[
            pl.BlockSpec((T_BLK, 1), lambda i, j: (i, 0)),
            pl.BlockSpec((T_BLK, 1), lambda i, j: (i, 0)),
        ],
        out_shape=[
            jax.ShapeDtypeStruct((N_TOK, 1), jnp.int32),
            jax.ShapeDtypeStruct((N_TOK, 1), jnp.float32),
        ],
    )(inputs, weight, sx, swb, colsb)
    return idx


_NW = 32
_TPW = N_TOK // _NW
_CH = 128
_NCH = _TPW // _CH


def _sc_body(idx_hbm, w_hbm, q_hbm, counts_hbm,
             idx_v, rows_v, ones_v, zero_v, shc, sem):
    cid = lax.axis_index("c")
    sid = lax.axis_index("s")
    wid = sid * 2 + cid
    base = wid * _NCH

    pltpu.sync_copy(idx_hbm.at[pl.ds(base, _NCH)], idx_v)
    copies = []
    for ch in range(_NCH):
        copies.append(pltpu.async_copy(
            w_hbm.at[idx_v.at[ch]], rows_v.at[pl.ds(ch * _CH, _CH)], sem))

    @pl.when(sid == 0)
    def _zero():
        def zbody(i, _):
            zero_v[pl.ds(i * 16, 16)] = jnp.zeros((16,), jnp.int32)
            return 0
        lax.fori_loop(0, N_EMB // 16, zbody, 0)
        pltpu.sync_copy(zero_v, shc)

    for k in range(8):
        ones_v[pl.ds(k * 16, 16)] = jnp.full((16,), 1, jnp.int32)

    plsc.subcore_barrier()
    for ch in range(_NCH):
        pltpu.sync_copy(ones_v, shc.at[idx_v.at[ch]], add=True)
    plsc.subcore_barrier()

    @pl.when(sid == 0)
    def _export():
        pltpu.sync_copy(shc, counts_hbm.at[cid])

    for c in copies:
        c.wait()
    pltpu.sync_copy(rows_v, q_hbm.at[pl.ds(wid * _TPW, _TPW)])


def _gather_hist_sc(idx2d, weight):
    mesh = plsc.VectorSubcoreMesh(core_axis_name="c", subcore_axis_name="s")
    fn = functools.partial(
        pl.kernel,
        mesh=mesh,
        out_type=[
            jax.ShapeDtypeStruct((N_TOK, D_EMB), jnp.float32),
            jax.ShapeDtypeStruct((2, N_EMB), jnp.int32),
        ],
        scratch_types=[
            pltpu.VMEM((_NCH, _CH), jnp.int32),
            pltpu.VMEM((_TPW, D_EMB), jnp.float32),
            pltpu.VMEM((_CH,), jnp.int32),
            pltpu.VMEM((N_EMB,), jnp.int32),
            pltpu.VMEM_SHARED((N_EMB,), jnp.int32),
            pltpu.SemaphoreType.DMA,
        ],
        compiler_params=pltpu.CompilerParams(use_tc_tiling_on_sc=False),
    )(_sc_body)
    return fn(idx2d, weight)


def _k3_body(x_ref, q_ref, cnt_ref, out_ref, loss_ref, perp_ref, acc_ref):
    b = pl.program_id(0)
    x = x_ref[0]
    qt = jnp.transpose(q_ref[...])
    diff = qt - x
    out_ref[...] = (x + diff)[None]
    part = jnp.sum(diff * diff)

    @pl.when(b == 0)
    def _init():
        acc_ref[0, 0] = part

    @pl.when(b > 0)
    def _acc():
        acc_ref[0, 0] = acc_ref[0, 0] + part

    @pl.when(b == 7)
    def _final():
        l = acc_ref[0, 0] * jnp.float32(1.0 / (N_TOK * D_EMB))
        loss_ref[...] = jnp.reshape(l + C_COST * l, (1, 1))
        ctot = cnt_ref[0:1, :] + cnt_ref[1:2, :]
        p = ctot.astype(jnp.float32) * jnp.float32(1.0 / N_TOK)
        ent = jnp.sum(p * jnp.log(p + 1e-10))
        perp_ref[...] = jnp.reshape(jnp.exp(-ent), (1, 1))


def _finish_tc(inputs, q, counts):
    return pl.pallas_call(
        _k3_body,
        grid=(8,),
        in_specs=[
            pl.BlockSpec((1, 32, 1024), lambda b: (b, 0, 0)),
            pl.BlockSpec((1024, D_EMB), lambda b: (b, 0)),
            pl.BlockSpec((2, N_EMB), lambda b: (0, 0)),
        ],
        out_specs=[
            pl.BlockSpec((1, 32, 1024), lambda b: (b, 0, 0)),
            pl.BlockSpec((1, 1), lambda b: (0, 0)),
            pl.BlockSpec((1, 1), lambda b: (0, 0)),
        ],
        out_shape=[
            jax.ShapeDtypeStruct((8, 32, 1024), jnp.float32),
            jax.ShapeDtypeStruct((1, 1), jnp.float32),
            jax.ShapeDtypeStruct((1, 1), jnp.float32),
        ],
        scratch_shapes=[pltpu.SMEM((1, 1), jnp.float32)],
    )(inputs, q, counts)


def kernel(inputs, weight):
    flat_x = jnp.transpose(inputs, (0, 2, 1)).reshape(N_TOK, D_EMB)
    sx = jnp.sum(flat_x * flat_x, axis=1).reshape(N_TOK, 1)
    swb = jnp.broadcast_to(jnp.sum(weight * weight, axis=1)[None, :], (8, N_EMB))
    colsb = jnp.broadcast_to(
        jnp.arange(N_EMB, dtype=jnp.float32)[None, :], (8, N_EMB))
    idx = _argmin_tc(inputs, weight + weight, sx, swb, colsb)
    q, counts = _gather_hist_sc(idx.reshape(_NW * _NCH, _CH), weight)
    quantized_out, loss, perp = _finish_tc(inputs, q, counts)
    enc_idx = idx.reshape(8, 1024)
    return (quantized_out, loss.reshape(()), enc_idx, perp.reshape(()))
